# baseline (speedup 1.0000x reference)
import functools

import jax
import jax.numpy as jnp
from jax.experimental import pallas as pl
from jax.experimental.pallas import tpu as pltpu

_BETA = 0.25
_NUM_CB = 4
_CB_N = 256
_E_DIM = 64


def _chain(h, w_refs, b_refs, last_relu=False):
    n = len(w_refs)
    for i in range(n):
        w = w_refs[i][...].astype(jnp.bfloat16)
        y = jnp.dot(h, w, preferred_element_type=jnp.float32) + b_refs[i][...]
        if i < n - 1 or last_relu:
            y = jnp.maximum(y, 0.0)
            h = y.astype(jnp.bfloat16)
    return y, h


def _enc2_body(n_blk, nbt, nbi, *refs):
    (xt_ref, w1t_ref, b1t_ref) = refs[0:3]
    wt_rest, bt_rest = refs[3:7], refs[7:11]
    (xi_ref, w1i_ref, b1i_ref) = refs[11:14]
    wi_rest, bi_rest = refs[14:18], refs[18:22]
    tcb_ref, tcbt_ref, icb_ref, icbt_ref = refs[22:26]
    (tzq_ref, tzqb_ref, tidx_ref, tloss_ref,
     izq_ref, izqb_ref, iidx_ref, iloss_ref) = refs[26:34]
    y1t_ref, y1i_ref = refs[34], refs[35]
    j = pl.program_id(0)

    w1t = w1t_ref[...].astype(jnp.bfloat16)
    yt = jnp.dot(xt_ref[...], w1t, preferred_element_type=jnp.float32)
    yt = jnp.maximum(yt + b1t_ref[...], 0.0)
    y1t_ref[:, pl.ds(j * nbt, nbt)] = yt.astype(jnp.bfloat16)

    w1i = w1i_ref[...].astype(jnp.bfloat16)
    yi = jnp.dot(xi_ref[...], w1i, preferred_element_type=jnp.float32)
    yi = jnp.maximum(yi + b1i_ref[...], 0.0)
    y1i_ref[:, pl.ds(j * nbi, nbi)] = yi.astype(jnp.bfloat16)

    @pl.when(j == n_blk - 1)
    def _():
        out_t, _ = _chain(y1t_ref[...], wt_rest, bt_rest)
        out_i, _ = _chain(y1i_ref[...], wi_rest, bi_rest)
        _rq_one(out_t, tcb_ref, tcbt_ref, tzq_ref, tzqb_ref, tidx_ref,
                tloss_ref)
        _rq_one(out_i, icb_ref, icbt_ref, izq_ref, izqb_ref, iidx_ref,
                iloss_ref)


def _mlp_specs(x, Ws, bs, nb):
    din = x.shape[1]
    specs = [
        pl.BlockSpec((x.shape[0], din), lambda j: (0, 0)),
        pl.BlockSpec((din, nb), lambda j: (0, j)),
        pl.BlockSpec((1, nb), lambda j: (0, j)),
    ]
    for W in Ws[1:]:
        specs.append(pl.BlockSpec(W.shape, lambda j, _s=W.shape: (0, 0)))
    for b in bs[1:]:
        specs.append(pl.BlockSpec((1, b.shape[-1]), lambda j: (0, 0)))
    return specs


def _mlp_args(x, Ws, bs):
    return [x, Ws[0], bs[0].reshape(1, -1), *Ws[1:],
            *[b.reshape(1, -1) for b in bs[1:]]]


def _run_enc2(xt_bf16, tWs, tbs, xi_bf16, iWs, ibs, t_cb, i_cb, nbt, nbi):
    B = xt_bf16.shape[0]
    d1 = tWs[0].shape[1]
    n_blk = d1 // nbt
    assert iWs[0].shape[1] // nbi == n_blk
    tcbt = jnp.transpose(t_cb, (0, 2, 1)).astype(jnp.bfloat16)
    icbt = jnp.transpose(i_cb, (0, 2, 1)).astype(jnp.bfloat16)
    in_specs = (_mlp_specs(xt_bf16, tWs, tbs, nbt)
                + _mlp_specs(xi_bf16, iWs, ibs, nbi))
    for arr in (t_cb, tcbt, i_cb, icbt):
        in_specs.append(pl.BlockSpec(arr.shape, lambda j, _s=arr.shape:
                                     (0,) * len(_s)))
    sds = jax.ShapeDtypeStruct
    const = lambda j: (0, 0)
    out_specs = (
        pl.BlockSpec((B, _E_DIM), const), pl.BlockSpec((B, _E_DIM), const),
        pl.BlockSpec((_NUM_CB, B), const), pl.BlockSpec((1, 1), const),
        pl.BlockSpec((B, _E_DIM), const), pl.BlockSpec((B, _E_DIM), const),
        pl.BlockSpec((_NUM_CB, B), const), pl.BlockSpec((1, 1), const),
    )
    out_shape = (
        sds((B, _E_DIM), jnp.float32), sds((B, _E_DIM), jnp.bfloat16),
        sds((_NUM_CB, B), jnp.int32), sds((1, 1), jnp.float32),
        sds((B, _E_DIM), jnp.float32), sds((B, _E_DIM), jnp.bfloat16),
        sds((_NUM_CB, B), jnp.int32), sds((1, 1), jnp.float32),
    )
    return pl.pallas_call(
        functools.partial(_enc2_body, n_blk, nbt, nbi),
        grid=(n_blk,),
        in_specs=in_specs,
        out_specs=out_specs,
        out_shape=out_shape,
        scratch_shapes=[pltpu.VMEM((B, d1), jnp.bfloat16),
                        pltpu.VMEM((B, iWs[0].shape[1]), jnp.bfloat16)],
    )(*_mlp_args(xt_bf16, tWs, tbs), *_mlp_args(xi_bf16, iWs, ibs),
      t_cb, tcbt, i_cb, icbt)


def _dec2_body(nb, *refs):
    zt_ref = refs[0]
    wt_rest, bt_rest = refs[1:5], refs[5:9]
    w5t_ref, b5t_ref = refs[9], refs[10]
    zi_ref = refs[11]
    wi_rest, bi_rest = refs[12:16], refs[16:20]
    w5i_ref, b5i_ref = refs[20], refs[21]
    outt_ref, outi_ref = refs[22], refs[23]
    y4t_ref, y4i_ref = refs[24], refs[25]
    j = pl.program_id(0)

    @pl.when(j == 0)
    def _():
        _, ht = _chain(zt_ref[...], wt_rest, bt_rest, last_relu=True)
        y4t_ref[...] = ht
        _, hi = _chain(zi_ref[...], wi_rest, bi_rest, last_relu=True)
        y4i_ref[...] = hi

    w5t = w5t_ref[...].astype(jnp.bfloat16)
    y = jnp.dot(y4t_ref[...], w5t, preferred_element_type=jnp.float32)
    outt_ref[...] = y + b5t_ref[...]

    @pl.when(j == 1)
    def _():
        w5i = w5i_ref[...].astype(jnp.bfloat16)
        yi = jnp.dot(y4i_ref[...], w5i, preferred_element_type=jnp.float32)
        outi_ref[...] = yi + b5i_ref[...]


def _run_dec2(zt_bf16, tWs, tbs, zi_bf16, iWs, ibs, nbt):
    B = zt_bf16.shape[0]
    d4 = tWs[-1].shape[0]
    dt, di = tWs[-1].shape[1], iWs[-1].shape[1]
    n_blk = dt // nbt
    sds = jax.ShapeDtypeStruct

    def half_specs(Ws, bs):
        specs = [pl.BlockSpec((B, _E_DIM), lambda j: (0, 0))]
        for W in Ws[:-1]:
            specs.append(pl.BlockSpec(W.shape, lambda j, _s=W.shape: (0, 0)))
        for b in bs[:-1]:
            specs.append(pl.BlockSpec((1, b.shape[-1]), lambda j: (0, 0)))
        return specs

    in_specs = half_specs(tWs, tbs)
    in_specs.append(pl.BlockSpec((d4, nbt), lambda j: (0, j)))
    in_specs.append(pl.BlockSpec((1, nbt), lambda j: (0, j)))
    in_specs += half_specs(iWs, ibs)
    in_specs.append(pl.BlockSpec((d4, di), lambda j: (0, 0)))
    in_specs.append(pl.BlockSpec((1, di), lambda j: (0, 0)))

    def half_args(z, Ws, bs):
        return [z, *Ws[:-1], *[b.reshape(1, -1) for b in bs[:-1]],
                Ws[-1], bs[-1].reshape(1, -1)]

    return pl.pallas_call(
        functools.partial(_dec2_body, nbt),
        grid=(n_blk,),
        in_specs=in_specs,
        out_specs=(pl.BlockSpec((B, nbt), lambda j: (0, j)),
                   pl.BlockSpec((B, di), lambda j: (0, 0))),
        out_shape=(sds((B, dt), jnp.float32), sds((B, di), jnp.float32)),
        scratch_shapes=[pltpu.VMEM((B, d4), jnp.bfloat16),
                        pltpu.VMEM((B, d4), jnp.bfloat16)],
    )(*half_args(zt_bf16, tWs, tbs), *half_args(zi_bf16, iWs, ibs))


def _dec_body(nb, *refs):
    z_ref = refs[0]
    w_rest = refs[1:5]
    b_rest = refs[5:9]
    w5_ref, b5_ref = refs[9], refs[10]
    out_ref = refs[11]
    y4_ref = refs[12]
    j = pl.program_id(0)

    @pl.when(j == 0)
    def _():
        _, h = _chain(z_ref[...], w_rest, b_rest, last_relu=True)
        y4_ref[...] = h

    w5 = w5_ref[...].astype(jnp.bfloat16)
    y = jnp.dot(y4_ref[...], w5, preferred_element_type=jnp.float32)
    out_ref[...] = y + b5_ref[...]


def _run_dec(z_bf16, Ws, bs, nb):
    B = z_bf16.shape[0]
    d4 = Ws[-1].shape[0]
    dout = Ws[-1].shape[1]
    n_blk = dout // nb
    in_specs = [pl.BlockSpec((B, _E_DIM), lambda j: (0, 0))]
    for W in Ws[:-1]:
        in_specs.append(pl.BlockSpec(W.shape, lambda j, _s=W.shape: (0, 0)))
    for b in bs[:-1]:
        in_specs.append(pl.BlockSpec((1, b.shape[-1]), lambda j: (0, 0)))
    in_specs.append(pl.BlockSpec((d4, nb), lambda j: (0, j)))
    in_specs.append(pl.BlockSpec((1, nb), lambda j: (0, j)))
    return pl.pallas_call(
        functools.partial(_dec_body, nb),
        grid=(n_blk,),
        in_specs=in_specs,
        out_specs=pl.BlockSpec((B, nb), lambda j: (0, j)),
        out_shape=jax.ShapeDtypeStruct((B, dout), jnp.float32),
        scratch_shapes=[pltpu.VMEM((B, d4), jnp.bfloat16)],
    )(z_bf16, *Ws[:-1], *[b.reshape(1, -1) for b in bs[:-1]],
      Ws[-1], bs[-1].reshape(1, -1))


def _rq_one(e, cb_ref, cbt_ref, zq_ref, zqb_ref, idx_ref, loss_ref):
    B = e.shape[0]
    r = e
    z = jnp.zeros_like(e)
    losses = []
    iota = jax.lax.broadcasted_iota(jnp.int32, (B, _CB_N), 1)
    for i in range(_NUM_CB):
        cb = cb_ref[i]
        cbt = cbt_ref[i]
        s = jnp.dot(r.astype(jnp.bfloat16), cbt,
                    preferred_element_type=jnp.float32)
        x2 = jnp.sum(r * r, axis=1, keepdims=True)
        c2 = jnp.sum(cb * cb, axis=1)[None, :]
        d = x2 + c2 - 2.0 * s
        m = jnp.min(d, axis=1, keepdims=True)
        idx = jnp.min(jnp.where(d == m, iota, _CB_N), axis=1)
        oh = (iota == idx[:, None]).astype(jnp.float32)
        xq = jax.lax.dot(oh, cb, precision=jax.lax.Precision.HIGHEST)
        l = jnp.mean((xq - r) ** 2)
        losses.append(l + _BETA * l)
        q = r + (xq - r)
        r = r - q
        z = z + q
        idx_ref[i, :] = idx
    zq_ref[...] = z
    zqb_ref[...] = z.astype(jnp.bfloat16)
    loss_ref[...] = jnp.mean(jnp.stack(losses)).reshape(1, 1)


def _rq2_body(te_ref, tcb_ref, tcbt_ref, ie_ref, icb_ref, icbt_ref,
              tzq_ref, tzqb_ref, tidx_ref, tloss_ref,
              izq_ref, izqb_ref, iidx_ref, iloss_ref):
    _rq_one(te_ref, tcb_ref, tcbt_ref, tzq_ref, tzqb_ref, tidx_ref, tloss_ref)
    _rq_one(ie_ref, icb_ref, icbt_ref, izq_ref, izqb_ref, iidx_ref, iloss_ref)


def _run_rq2(te, t_cb, ie, i_cb):
    B = te.shape[0]
    tcbt = jnp.transpose(t_cb, (0, 2, 1)).astype(jnp.bfloat16)
    icbt = jnp.transpose(i_cb, (0, 2, 1)).astype(jnp.bfloat16)
    sds = jax.ShapeDtypeStruct
    out_shapes = (
        sds((B, _E_DIM), jnp.float32), sds((B, _E_DIM), jnp.bfloat16),
        sds((_NUM_CB, B), jnp.int32), sds((1, 1), jnp.float32),
        sds((B, _E_DIM), jnp.float32), sds((B, _E_DIM), jnp.bfloat16),
        sds((_NUM_CB, B), jnp.int32), sds((1, 1), jnp.float32),
    )
    tzq, tzqb, tidx, tloss, izq, izqb, iidx, iloss = pl.pallas_call(
        _rq2_body,
        out_shape=out_shapes,
    )(te, t_cb, tcbt, ie, i_cb, icbt)
    return ((tzq, tzqb, tidx.T, tloss.reshape(())),
            (izq, izqb, iidx.T, iloss.reshape(())))


def kernel(text_x, image_x, params):
    t_enc_W, t_enc_b = params['t_enc_W'], params['t_enc_b']
    t_dec_W, t_dec_b = params['t_dec_W'], params['t_dec_b']
    i_enc_W, i_enc_b = params['i_enc_W'], params['i_enc_b']
    i_dec_W, i_dec_b = params['i_dec_W'], params['i_dec_b']
    t_cb = jnp.stack(params['t_cb'])
    i_cb = jnp.stack(params['i_cb'])

    (z_q_text, zqt_b, tidx_r, text_loss_a,
     z_q_image, zqi_b, iidx_r, image_loss_a) = _run_enc2(
        text_x.astype(jnp.bfloat16), t_enc_W, t_enc_b,
        image_x.astype(jnp.bfloat16), i_enc_W, i_enc_b, t_cb, i_cb, 128, 128)
    text_idx, image_idx = tidx_r.T, iidx_r.T
    text_loss = text_loss_a.reshape(())
    image_loss = image_loss_a.reshape(())

    text_out = _run_dec(zqt_b, t_dec_W, t_dec_b, 512)
    image_out = _run_dec(zqi_b, i_dec_W, i_dec_b, 256)

    return (text_out, image_out, text_loss, image_loss,
            text_idx, image_idx, z_q_text, z_q_image)


# speedup vs baseline: 1.1773x; 1.1773x over previous
import functools

import jax
import jax.numpy as jnp
from jax.experimental import pallas as pl
from jax.experimental.pallas import tpu as pltpu

_BETA = 0.25
_NUM_CB = 4
_CB_N = 256
_E_DIM = 64


def _chain(h, w_refs, b_refs, last_relu=False):
    n = len(w_refs)
    for i in range(n):
        w = w_refs[i][...].astype(jnp.bfloat16)
        y = jnp.dot(h, w, preferred_element_type=jnp.float32) + b_refs[i][...]
        if i < n - 1 or last_relu:
            y = jnp.maximum(y, 0.0)
            h = y.astype(jnp.bfloat16)
    return y, h


def _enc2_body(n_blk, nbt, nbi, *refs):
    (xt_ref, w1t_ref, b1t_ref) = refs[0:3]
    wt_rest, bt_rest = refs[3:7], refs[7:11]
    (xi_ref, w1i_ref, b1i_ref) = refs[11:14]
    wi_rest, bi_rest = refs[14:18], refs[18:22]
    outt_ref, outi_ref = refs[22], refs[23]
    y1t_ref, y1i_ref = refs[24], refs[25]
    j = pl.program_id(0)

    w1t = w1t_ref[...].astype(jnp.bfloat16)
    yt = jnp.dot(xt_ref[...], w1t, preferred_element_type=jnp.float32)
    yt = jnp.maximum(yt + b1t_ref[...], 0.0)
    y1t_ref[:, pl.ds(j * nbt, nbt)] = yt.astype(jnp.bfloat16)

    w1i = w1i_ref[...].astype(jnp.bfloat16)
    yi = jnp.dot(xi_ref[...], w1i, preferred_element_type=jnp.float32)
    yi = jnp.maximum(yi + b1i_ref[...], 0.0)
    y1i_ref[:, pl.ds(j * nbi, nbi)] = yi.astype(jnp.bfloat16)

    @pl.when(j == n_blk - 1)
    def _():
        out_t, _ = _chain(y1t_ref[...], wt_rest, bt_rest)
        out_i, _ = _chain(y1i_ref[...], wi_rest, bi_rest)
        outt_ref[...] = out_t
        outi_ref[...] = out_i


def _mlp_specs(x, Ws, bs, nb):
    din = x.shape[1]
    specs = [
        pl.BlockSpec((x.shape[0], din), lambda j: (0, 0)),
        pl.BlockSpec((din, nb), lambda j: (0, j)),
        pl.BlockSpec((1, nb), lambda j: (0, j)),
    ]
    for W in Ws[1:]:
        specs.append(pl.BlockSpec(W.shape, lambda j, _s=W.shape: (0, 0)))
    for b in bs[1:]:
        specs.append(pl.BlockSpec((1, b.shape[-1]), lambda j: (0, 0)))
    return specs


def _mlp_args(x, Ws, bs):
    return [x, Ws[0], bs[0].reshape(1, -1), *Ws[1:],
            *[b.reshape(1, -1) for b in bs[1:]]]


def _run_enc2(xt_bf16, tWs, tbs, xi_bf16, iWs, ibs, nbt, nbi):
    B = xt_bf16.shape[0]
    d1 = tWs[0].shape[1]
    n_blk = d1 // nbt
    assert iWs[0].shape[1] // nbi == n_blk
    in_specs = (_mlp_specs(xt_bf16, tWs, tbs, nbt)
                + _mlp_specs(xi_bf16, iWs, ibs, nbi))
    sds = jax.ShapeDtypeStruct
    return pl.pallas_call(
        functools.partial(_enc2_body, n_blk, nbt, nbi),
        grid=(n_blk,),
        in_specs=in_specs,
        out_specs=(pl.BlockSpec((B, _E_DIM), lambda j: (0, 0)),
                   pl.BlockSpec((B, _E_DIM), lambda j: (0, 0))),
        out_shape=(sds((B, _E_DIM), jnp.float32),
                   sds((B, _E_DIM), jnp.float32)),
        scratch_shapes=[pltpu.VMEM((B, d1), jnp.bfloat16),
                        pltpu.VMEM((B, iWs[0].shape[1]), jnp.bfloat16)],
    )(*_mlp_args(xt_bf16, tWs, tbs), *_mlp_args(xi_bf16, iWs, ibs))


def _dec2_body(nb, *refs):
    zt_ref = refs[0]
    wt_rest, bt_rest = refs[1:5], refs[5:9]
    w5t_ref, b5t_ref = refs[9], refs[10]
    zi_ref = refs[11]
    wi_rest, bi_rest = refs[12:16], refs[16:20]
    w5i_ref, b5i_ref = refs[20], refs[21]
    outt_ref, outi_ref = refs[22], refs[23]
    y4t_ref, y4i_ref = refs[24], refs[25]
    j = pl.program_id(0)

    @pl.when(j == 0)
    def _():
        _, ht = _chain(zt_ref[...], wt_rest, bt_rest, last_relu=True)
        y4t_ref[...] = ht
        _, hi = _chain(zi_ref[...], wi_rest, bi_rest, last_relu=True)
        y4i_ref[...] = hi

    w5t = w5t_ref[...].astype(jnp.bfloat16)
    y = jnp.dot(y4t_ref[...], w5t, preferred_element_type=jnp.float32)
    outt_ref[...] = y + b5t_ref[...]

    @pl.when(j == 1)
    def _():
        w5i = w5i_ref[...].astype(jnp.bfloat16)
        yi = jnp.dot(y4i_ref[...], w5i, preferred_element_type=jnp.float32)
        outi_ref[...] = yi + b5i_ref[...]


def _run_dec2(zt_bf16, tWs, tbs, zi_bf16, iWs, ibs, nbt):
    B = zt_bf16.shape[0]
    d4 = tWs[-1].shape[0]
    dt, di = tWs[-1].shape[1], iWs[-1].shape[1]
    n_blk = dt // nbt
    sds = jax.ShapeDtypeStruct

    def half_specs(Ws, bs):
        specs = [pl.BlockSpec((B, _E_DIM), lambda j: (0, 0))]
        for W in Ws[:-1]:
            specs.append(pl.BlockSpec(W.shape, lambda j, _s=W.shape: (0, 0)))
        for b in bs[:-1]:
            specs.append(pl.BlockSpec((1, b.shape[-1]), lambda j: (0, 0)))
        return specs

    in_specs = half_specs(tWs, tbs)
    in_specs.append(pl.BlockSpec((d4, nbt), lambda j: (0, j)))
    in_specs.append(pl.BlockSpec((1, nbt), lambda j: (0, j)))
    in_specs += half_specs(iWs, ibs)
    in_specs.append(pl.BlockSpec((d4, di), lambda j: (0, 0)))
    in_specs.append(pl.BlockSpec((1, di), lambda j: (0, 0)))

    def half_args(z, Ws, bs):
        return [z, *Ws[:-1], *[b.reshape(1, -1) for b in bs[:-1]],
                Ws[-1], bs[-1].reshape(1, -1)]

    return pl.pallas_call(
        functools.partial(_dec2_body, nbt),
        grid=(n_blk,),
        in_specs=in_specs,
        out_specs=(pl.BlockSpec((B, nbt), lambda j: (0, j)),
                   pl.BlockSpec((B, di), lambda j: (0, 0))),
        out_shape=(sds((B, dt), jnp.float32), sds((B, di), jnp.float32)),
        scratch_shapes=[pltpu.VMEM((B, d4), jnp.bfloat16),
                        pltpu.VMEM((B, d4), jnp.bfloat16)],
    )(*half_args(zt_bf16, tWs, tbs), *half_args(zi_bf16, iWs, ibs))


def _dec_body(nb, *refs):
    z_ref = refs[0]
    w_rest = refs[1:5]
    b_rest = refs[5:9]
    w5_ref, b5_ref = refs[9], refs[10]
    out_ref = refs[11]
    y4_ref = refs[12]
    j = pl.program_id(0)

    @pl.when(j == 0)
    def _():
        _, h = _chain(z_ref[...], w_rest, b_rest, last_relu=True)
        y4_ref[...] = h

    w5 = w5_ref[...].astype(jnp.bfloat16)
    y = jnp.dot(y4_ref[...], w5, preferred_element_type=jnp.float32)
    out_ref[...] = y + b5_ref[...]


def _run_dec(z_bf16, Ws, bs, nb):
    B = z_bf16.shape[0]
    d4 = Ws[-1].shape[0]
    dout = Ws[-1].shape[1]
    n_blk = dout // nb
    in_specs = [pl.BlockSpec((B, _E_DIM), lambda j: (0, 0))]
    for W in Ws[:-1]:
        in_specs.append(pl.BlockSpec(W.shape, lambda j, _s=W.shape: (0, 0)))
    for b in bs[:-1]:
        in_specs.append(pl.BlockSpec((1, b.shape[-1]), lambda j: (0, 0)))
    in_specs.append(pl.BlockSpec((d4, nb), lambda j: (0, j)))
    in_specs.append(pl.BlockSpec((1, nb), lambda j: (0, j)))
    return pl.pallas_call(
        functools.partial(_dec_body, nb),
        grid=(n_blk,),
        in_specs=in_specs,
        out_specs=pl.BlockSpec((B, nb), lambda j: (0, j)),
        out_shape=jax.ShapeDtypeStruct((B, dout), jnp.float32),
        scratch_shapes=[pltpu.VMEM((B, d4), jnp.bfloat16)],
    )(z_bf16, *Ws[:-1], *[b.reshape(1, -1) for b in bs[:-1]],
      Ws[-1], bs[-1].reshape(1, -1))


def _rq_one(e, cb_ref, cbt_ref, zq_ref, zqb_ref, idx_ref, loss_ref):
    B = e.shape[0]
    r = e
    z = jnp.zeros_like(e)
    losses = []
    iota = jax.lax.broadcasted_iota(jnp.int32, (B, _CB_N), 1)
    for i in range(_NUM_CB):
        cb = cb_ref[i]
        cbt = cbt_ref[i]
        s = jnp.dot(r.astype(jnp.bfloat16), cbt,
                    preferred_element_type=jnp.float32)
        x2 = jnp.sum(r * r, axis=1, keepdims=True)
        c2 = jnp.sum(cb * cb, axis=1)[None, :]
        d = x2 + c2 - 2.0 * s
        m = jnp.min(d, axis=1, keepdims=True)
        idx = jnp.min(jnp.where(d == m, iota, _CB_N), axis=1)
        oh = (iota == idx[:, None]).astype(jnp.float32)
        xq = jax.lax.dot(oh, cb, precision=jax.lax.Precision.HIGHEST)
        l = jnp.mean((xq - r) ** 2)
        losses.append(l + _BETA * l)
        q = r + (xq - r)
        r = r - q
        z = z + q
        idx_ref[i, :] = idx
    zq_ref[...] = z
    zqb_ref[...] = z.astype(jnp.bfloat16)
    loss_ref[...] = jnp.mean(jnp.stack(losses)).reshape(1, 1)


def _rq2_body(te_ref, tcb_ref, tcbt_ref, ie_ref, icb_ref, icbt_ref,
              tzq_ref, tzqb_ref, tidx_ref, tloss_ref,
              izq_ref, izqb_ref, iidx_ref, iloss_ref):
    _rq_one(te_ref[...], tcb_ref, tcbt_ref, tzq_ref, tzqb_ref, tidx_ref,
            tloss_ref)
    _rq_one(ie_ref[...], icb_ref, icbt_ref, izq_ref, izqb_ref, iidx_ref,
            iloss_ref)


def _run_rq2(te, t_cb, ie, i_cb):
    B = te.shape[0]
    tcbt = jnp.transpose(t_cb, (0, 2, 1)).astype(jnp.bfloat16)
    icbt = jnp.transpose(i_cb, (0, 2, 1)).astype(jnp.bfloat16)
    sds = jax.ShapeDtypeStruct
    out_shapes = (
        sds((B, _E_DIM), jnp.float32), sds((B, _E_DIM), jnp.bfloat16),
        sds((_NUM_CB, B), jnp.int32), sds((1, 1), jnp.float32),
        sds((B, _E_DIM), jnp.float32), sds((B, _E_DIM), jnp.bfloat16),
        sds((_NUM_CB, B), jnp.int32), sds((1, 1), jnp.float32),
    )
    tzq, tzqb, tidx, tloss, izq, izqb, iidx, iloss = pl.pallas_call(
        _rq2_body,
        out_shape=out_shapes,
    )(te, t_cb, tcbt, ie, i_cb, icbt)
    return ((tzq, tzqb, tidx.T, tloss.reshape(())),
            (izq, izqb, iidx.T, iloss.reshape(())))


def kernel(text_x, image_x, params):
    t_enc_W, t_enc_b = params['t_enc_W'], params['t_enc_b']
    t_dec_W, t_dec_b = params['t_dec_W'], params['t_dec_b']
    i_enc_W, i_enc_b = params['i_enc_W'], params['i_enc_b']
    i_dec_W, i_dec_b = params['i_dec_W'], params['i_dec_b']
    t_cb = jnp.stack(params['t_cb'])
    i_cb = jnp.stack(params['i_cb'])

    text_e, image_e = _run_enc2(
        text_x.astype(jnp.bfloat16), t_enc_W, t_enc_b,
        image_x.astype(jnp.bfloat16), i_enc_W, i_enc_b, 256, 256)

    ((z_q_text, zqt_b, text_idx, text_loss),
     (z_q_image, zqi_b, image_idx, image_loss)) = _run_rq2(
        text_e, t_cb, image_e, i_cb)

    text_out = _run_dec(zqt_b, t_dec_W, t_dec_b, 512)
    image_out = _run_dec(zqi_b, i_dec_W, i_dec_b, 256)

    return (text_out, image_out, text_loss, image_loss,
            text_idx, image_idx, z_q_text, z_q_image)
